# Initial kernel scaffold; baseline (speedup 1.0000x reference)
import jax
import jax.numpy as jnp
from jax.experimental import pallas as pl


def kernel(x, edge_index, edge_weights, W1, W2, Wc, gn1_alpha, gn1_gamma, gn1_beta, gn2_alpha, gn2_gamma, gn2_beta):
    raise NotImplementedError("write your pallas kernel here")


# speedup vs baseline: 2.9880x; 2.9880x over previous
import functools

import jax
import jax.numpy as jnp
from jax import lax
from jax.experimental import pallas as pl
from jax.experimental.pallas import tpu as pltpu
from jax.experimental.pallas import tpu_sc as plsc

N = 10000
NPAD = 10240
E = 320000
D1 = 128
D2 = 64
OUT_DIM = 10
NCORES = 2
NSUB = 16
NW = NCORES * NSUB
EPT = 10240
EPAD = NW * EPT
CB = 128
NCHUNK = EPT // CB
ROWS = NPAD // 16
SLAB = NPAD // NSUB

_f32 = jnp.float32
_i32 = jnp.int32

_sc_mesh = plsc.VectorSubcoreMesh(core_axis_name="c", subcore_axis_name="s")
_sc_params = pltpu.CompilerParams(needs_layout_passes=False,
                                  use_tc_tiling_on_sc=False)


def _deg_body(src_hbm, dst_hbm, out_hbm, hs, hd, idx_s, idx_d):
    c = lax.axis_index("c")
    s = lax.axis_index("s")
    wid = c * NSUB + s

    def zloop(j, _):
        hs[pl.ds(j * 16, 16)] = jnp.zeros((16,), _f32)
        hd[pl.ds(j * 16, 16)] = jnp.zeros((16,), _f32)
        return 0
    lax.fori_loop(0, NPAD // 16, zloop, 0)

    ones = jnp.full((16,), 1.0, _f32)
    CA = 1024
    base = wid * EPT

    def chunk(k, _):
        pltpu.sync_copy(src_hbm.at[pl.ds(base + k * CA, CA)], idx_s)
        pltpu.sync_copy(dst_hbm.at[pl.ds(base + k * CA, CA)], idx_d)

        def inner(j, _):
            sv = idx_s[pl.ds(j * 16, 16)]
            dv = idx_d[pl.ds(j * 16, 16)]
            plsc.addupdate_scatter(hs, [sv], ones)
            plsc.addupdate_scatter(hd, [dv], ones)
            return 0
        lax.fori_loop(0, CA // 16, inner, 0)
        return 0
    lax.fori_loop(0, EPT // CA, chunk, 0)

    pltpu.sync_copy(hs, out_hbm.at[wid, 0])
    pltpu.sync_copy(hd, out_hbm.at[wid, 1])


def _deg_call(srcp, dstp):
    return pl.kernel(
        _deg_body,
        out_type=jax.ShapeDtypeStruct((NW, 2, NPAD), _f32),
        mesh=_sc_mesh,
        compiler_params=_sc_params,
        scratch_types=[
            pltpu.VMEM((NPAD,), _f32),
            pltpu.VMEM((NPAD,), _f32),
            pltpu.VMEM((1024,), _i32),
            pltpu.VMEM((1024,), _i32),
        ],
    )(srcp, dstp)


def _msg_body(D, z_hbm, src_hbm, dst_hbm, ew_hbm, no_hbm, ni_hbm, out_hbm,
              no_v, ni_v, sidx, didx, ewv, wv, rows, gsem, agg):
    c = lax.axis_index("c")
    s = lax.axis_index("s")
    wid = c * NSUB + s
    nq = D // 16

    pltpu.sync_copy(no_hbm, no_v)
    pltpu.sync_copy(ni_hbm, ni_v)

    def zloop(j, _):
        for q in range(nq):
            rows[j, pl.ds(q * 16, 16)] = jnp.zeros((16,), _f32)
        return 0
    lax.fori_loop(0, CB, zloop, 0)
    for t in range(SLAB // CB):
        pltpu.sync_copy(rows, agg.at[pl.ds(s * SLAB + t * CB, CB)])
    plsc.subcore_barrier()

    base = wid * EPT

    def chunk(k, _):
        off = base + k * CB
        pltpu.sync_copy(src_hbm.at[pl.ds(off, CB)], sidx)
        pltpu.sync_copy(dst_hbm.at[pl.ds(off, CB)], didx)
        pltpu.sync_copy(ew_hbm.at[pl.ds(off, CB)], ewv)
        pltpu.async_copy(z_hbm.at[sidx], rows, gsem).wait()

        def wcomp(j, _):
            sv = sidx[pl.ds(j * 16, 16)]
            dv = didx[pl.ds(j * 16, 16)]
            nov = plsc.load_gather(no_v, [sv])
            niv = plsc.load_gather(ni_v, [dv])
            wv[pl.ds(j * 16, 16)] = ewv[pl.ds(j * 16, 16)] * nov * niv
            return 0
        lax.fori_loop(0, CB // 16, wcomp, 0)

        def scale(e, _):
            ev = jnp.zeros((16,), _i32) + e
            w16 = plsc.load_gather(wv, [ev])
            for q in range(nq):
                rows[e, pl.ds(q * 16, 16)] = rows[e, pl.ds(q * 16, 16)] * w16
            return 0
        lax.fori_loop(0, CB, scale, 0)

        pltpu.sync_copy(rows, agg.at[didx], add=True)
        return 0
    lax.fori_loop(0, NCHUNK, chunk, 0)

    plsc.subcore_barrier()
    for t in range(SLAB // CB):
        pltpu.sync_copy(agg.at[pl.ds(s * SLAB + t * CB, CB)],
                        out_hbm.at[c, pl.ds(s * SLAB + t * CB, CB)])


def _msg_call(z, srcp, dstp, ewp, no, ni, D):
    return pl.kernel(
        functools.partial(_msg_body, D),
        out_type=jax.ShapeDtypeStruct((NCORES, NPAD, D), _f32),
        mesh=_sc_mesh,
        compiler_params=_sc_params,
        scratch_types=[
            pltpu.VMEM((NPAD,), _f32),
            pltpu.VMEM((NPAD,), _f32),
            pltpu.VMEM((CB,), _i32),
            pltpu.VMEM((CB,), _i32),
            pltpu.VMEM((CB,), _f32),
            pltpu.VMEM((CB,), _f32),
            pltpu.VMEM((CB, D), _f32),
            pltpu.SemaphoreType.DMA,
            pltpu.VMEM_SHARED((NPAD, D), _f32),
        ],
    )(z, srcp, dstp, ewp, no, ni)


def _norm_kernel(dref, oref):
    d = jnp.sum(dref[...], axis=0)
    oref[...] = lax.rsqrt(jnp.clip(d, 1.0, None))


def _mm_kernel(xref, wref, oref):
    oref[...] = jnp.dot(xref[...], wref[...], preferred_element_type=_f32)


def _stats_kernel(p0, p1, aref, sref):
    i = pl.program_id(0)
    a = p0[...] + p1[...]
    a = jnp.where(a >= 0.0, a, 0.01 * a)
    aref[...] = a

    @pl.when(i == 0)
    def _():
        sref[...] = jnp.zeros_like(sref)

    sref[0:1, :] = sref[0:1, :] + jnp.sum(a, axis=0, keepdims=True)
    sref[1:2, :] = sref[1:2, :] + jnp.sum(a * a, axis=0, keepdims=True)


def _gnmm_kernel(aref, sref, gref, alref, bref, wref, oref):
    inv_n = jnp.float32(1.0 / N)
    mu = sref[0:1, :] * inv_n
    e2 = sref[1:2, :] * inv_n
    al = alref[...]
    var = e2 - (2.0 * al - al * al) * mu * mu
    sc = gref[...] * lax.rsqrt(var + 1e-5)
    t = bref[...] - sc * al * mu
    h = aref[...] * sc + t
    oref[...] = jnp.dot(h, wref[...], preferred_element_type=_f32)


def _final_kernel(p0, p1, gref, alref, bref, wcref, oref, acc):
    i = pl.program_id(0)
    nb = NPAD // 256
    a = p0[...] + p1[...]
    a = jnp.where(a >= 0.0, a, 0.01 * a)
    rid = i * 256 + lax.broadcasted_iota(_i32, (256, 1), 0)
    valid = rid < N
    big = jnp.float32(3.0e38)
    amax = jnp.where(valid, a, -big)
    amin = jnp.where(valid, a, big)

    @pl.when(i == 0)
    def _():
        acc[0:2, :] = jnp.zeros((2, D2), _f32)
        acc[2:3, :] = jnp.full((1, D2), -big, _f32)
        acc[3:4, :] = jnp.full((1, D2), big, _f32)

    acc[0:1, :] = acc[0:1, :] + jnp.sum(a, axis=0, keepdims=True)
    acc[1:2, :] = acc[1:2, :] + jnp.sum(a * a, axis=0, keepdims=True)
    acc[2:3, :] = jnp.maximum(acc[2:3, :], jnp.max(amax, axis=0, keepdims=True))
    acc[3:4, :] = jnp.minimum(acc[3:4, :], jnp.min(amin, axis=0, keepdims=True))

    @pl.when(i == nb - 1)
    def _():
        inv_n = jnp.float32(1.0 / N)
        mu = acc[0:1, :] * inv_n
        e2 = acc[1:2, :] * inv_n
        al = alref[...]
        var = e2 - (2.0 * al - al * al) * mu * mu
        sc = gref[...] * lax.rsqrt(var + 1e-5)
        t = bref[...] - sc * al * mu
        pos = sc >= 0.0
        meanh = sc * mu + t
        maxh = jnp.where(pos, sc * acc[2:3, :], sc * acc[3:4, :]) + t
        minh = jnp.where(pos, sc * acc[3:4, :], sc * acc[2:3, :]) + t
        feats = jnp.concatenate([meanh, maxh, minh], axis=1)
        oref[...] = jnp.dot(feats, wcref[...], preferred_element_type=_f32)


def _row_spec(d):
    return pl.BlockSpec((256, d), lambda i: (i, 0))


def _const_spec(shape):
    return pl.BlockSpec(shape, lambda i: tuple(0 for _ in shape))


def kernel(x, edge_index, edge_weights, W1, W2, Wc,
           gn1_alpha, gn1_gamma, gn1_beta,
           gn2_alpha, gn2_gamma, gn2_beta):
    src = edge_index[0].astype(_i32)
    dst = edge_index[1].astype(_i32)
    padi = jnp.full((EPAD - E,), N, _i32)
    srcp = jnp.concatenate([src, padi])
    dstp = jnp.concatenate([dst, padi])
    ewp = jnp.concatenate([edge_weights.astype(_f32),
                           jnp.zeros((EPAD - E,), _f32)])
    xp = jnp.pad(x, ((0, NPAD - N), (0, 0)))

    degs = _deg_call(srcp, dstp)
    norms = pl.pallas_call(
        _norm_kernel,
        out_shape=jax.ShapeDtypeStruct((2, NPAD // 128, 128), _f32),
    )(degs.reshape(NW, 2, NPAD // 128, 128))
    no = norms[0].reshape(NPAD)
    ni = norms[1].reshape(NPAD)

    nb = NPAD // 256
    z1 = pl.pallas_call(
        _mm_kernel,
        grid=(nb,),
        in_specs=[_row_spec(D1), _const_spec((D1, D1))],
        out_specs=_row_spec(D1),
        out_shape=jax.ShapeDtypeStruct((NPAD, D1), _f32),
    )(xp, W1)

    p1 = _msg_call(z1, srcp, dstp, ewp, no, ni, D1)

    a1, st1 = pl.pallas_call(
        _stats_kernel,
        grid=(nb,),
        in_specs=[_row_spec(D1), _row_spec(D1)],
        out_specs=[_row_spec(D1), _const_spec((8, D1))],
        out_shape=[jax.ShapeDtypeStruct((NPAD, D1), _f32),
                   jax.ShapeDtypeStruct((8, D1), _f32)],
    )(p1[0], p1[1])

    z2 = pl.pallas_call(
        _gnmm_kernel,
        grid=(nb,),
        in_specs=[_row_spec(D1), _const_spec((8, D1)), _const_spec((1, D1)),
                  _const_spec((1, D1)), _const_spec((1, D1)),
                  _const_spec((D1, D2))],
        out_specs=_row_spec(D2),
        out_shape=jax.ShapeDtypeStruct((NPAD, D2), _f32),
    )(a1, st1, gn1_gamma.reshape(1, D1), gn1_alpha.reshape(1, D1),
      gn1_beta.reshape(1, D1), W2)

    p2 = _msg_call(z2, srcp, dstp, ewp, no, ni, D2)

    out = pl.pallas_call(
        _final_kernel,
        grid=(nb,),
        in_specs=[_row_spec(D2), _row_spec(D2), _const_spec((1, D2)),
                  _const_spec((1, D2)), _const_spec((1, D2)),
                  _const_spec((3 * D2, OUT_DIM))],
        out_specs=_const_spec((1, OUT_DIM)),
        out_shape=jax.ShapeDtypeStruct((1, OUT_DIM), _f32),
        scratch_shapes=[pltpu.VMEM((8, D2), _f32)],
    )(p2[0], p2[1], gn2_gamma.reshape(1, D2), gn2_alpha.reshape(1, D2),
      gn2_beta.reshape(1, D2), Wc)

    return out


# speedup vs baseline: 4.3153x; 1.4442x over previous
import functools

import jax
import jax.numpy as jnp
from jax import lax
from jax.experimental import pallas as pl
from jax.experimental.pallas import tpu as pltpu
from jax.experimental.pallas import tpu_sc as plsc

N = 10000
NPAD = 10240
E = 320000
D1 = 128
D2 = 64
OUT_DIM = 10
NCORES = 2
NSUB = 16
NW = NCORES * NSUB
EPT = 10240
EPAD = NW * EPT
CB = 64
NCHUNK = EPT // CB
ROWS = NPAD // 16
SLAB = NPAD // NSUB

_f32 = jnp.float32
_i32 = jnp.int32

_sc_mesh = plsc.VectorSubcoreMesh(core_axis_name="c", subcore_axis_name="s")
_sc_params = pltpu.CompilerParams(needs_layout_passes=False,
                                  use_tc_tiling_on_sc=False)


def _deg_body(src_hbm, dst_hbm, out_hbm, hs, hd, idx_s, idx_d):
    c = lax.axis_index("c")
    s = lax.axis_index("s")
    wid = c * NSUB + s

    def zloop(j, _):
        hs[pl.ds(j * 16, 16)] = jnp.zeros((16,), _f32)
        hd[pl.ds(j * 16, 16)] = jnp.zeros((16,), _f32)
        return 0
    lax.fori_loop(0, NPAD // 16, zloop, 0)

    ones = jnp.full((16,), 1.0, _f32)
    CA = 1024
    base = wid * EPT

    def chunk(k, _):
        pltpu.sync_copy(src_hbm.at[pl.ds(base + k * CA, CA)], idx_s)
        pltpu.sync_copy(dst_hbm.at[pl.ds(base + k * CA, CA)], idx_d)

        def inner(j, _):
            sv = idx_s[pl.ds(j * 16, 16)]
            dv = idx_d[pl.ds(j * 16, 16)]
            plsc.addupdate_scatter(hs, [sv], ones)
            plsc.addupdate_scatter(hd, [dv], ones)
            return 0
        lax.fori_loop(0, CA // 16, inner, 0)
        return 0
    lax.fori_loop(0, EPT // CA, chunk, 0)

    pltpu.sync_copy(hs, out_hbm.at[wid, 0])
    pltpu.sync_copy(hd, out_hbm.at[wid, 1])


def _deg_call(srcp, dstp):
    return pl.kernel(
        _deg_body,
        out_type=jax.ShapeDtypeStruct((NW, 2, NPAD), _f32),
        mesh=_sc_mesh,
        compiler_params=_sc_params,
        scratch_types=[
            pltpu.VMEM((NPAD,), _f32),
            pltpu.VMEM((NPAD,), _f32),
            pltpu.VMEM((1024,), _i32),
            pltpu.VMEM((1024,), _i32),
        ],
    )(srcp, dstp)


def _w_body(src_hbm, dst_hbm, ew_hbm, no_hbm, ni_hbm, out_hbm,
            no_v, ni_v, s_all, d_all, e_all, w_all):
    c = lax.axis_index("c")
    s = lax.axis_index("s")
    wid = c * NSUB + s
    base = wid * EPT

    pltpu.sync_copy(no_hbm, no_v)
    pltpu.sync_copy(ni_hbm, ni_v)
    pltpu.sync_copy(src_hbm.at[pl.ds(base, EPT)], s_all)
    pltpu.sync_copy(dst_hbm.at[pl.ds(base, EPT)], d_all)
    pltpu.sync_copy(ew_hbm.at[pl.ds(base, EPT)], e_all)

    def wcomp(j, _):
        sv = s_all[pl.ds(j * 16, 16)]
        dv = d_all[pl.ds(j * 16, 16)]
        nov = plsc.load_gather(no_v, [sv])
        niv = plsc.load_gather(ni_v, [dv])
        w_all[pl.ds(j * 16, 16)] = e_all[pl.ds(j * 16, 16)] * nov * niv
        return 0
    lax.fori_loop(0, EPT // 16, wcomp, 0)

    pltpu.sync_copy(w_all, out_hbm.at[pl.ds(base, EPT)])


def _w_call(srcp, dstp, ewp, no, ni):
    return pl.kernel(
        _w_body,
        out_type=jax.ShapeDtypeStruct((EPAD,), _f32),
        mesh=_sc_mesh,
        compiler_params=_sc_params,
        scratch_types=[
            pltpu.VMEM((NPAD,), _f32),
            pltpu.VMEM((NPAD,), _f32),
            pltpu.VMEM((EPT,), _i32),
            pltpu.VMEM((EPT,), _i32),
            pltpu.VMEM((EPT,), _f32),
            pltpu.VMEM((EPT,), _f32),
        ],
    )(srcp, dstp, ewp, no, ni)


def _msg_body(D, z_hbm, src3, dst3, wf_hbm, out_hbm,
              sall, dring, wring,
              rows0, rows1, rows2, rows3,
              sg0, sg1, sg2, sg3, ss0, ss1, ss2, ss3,
              sd0, sd1, sd2, sd3, sw0, sw1, sw2, sw3, agg):
    c = lax.axis_index("c")
    s = lax.axis_index("s")
    wid = c * NSUB + s
    nq = D // 16
    rows_l = (rows0, rows1, rows2, rows3)
    sg_l = (sg0, sg1, sg2, sg3)
    ss_l = (ss0, ss1, ss2, ss3)
    sd_l = (sd0, sd1, sd2, sd3)
    sw_l = (sw0, sw1, sw2, sw3)
    wbase = wid * EPT

    pltpu.sync_copy(src3.at[wid], sall)

    def zloop(j, _):
        for q in range(nq):
            rows0[j, pl.ds(q * 16, 16)] = jnp.zeros((16,), _f32)
        return 0
    lax.fori_loop(0, CB, zloop, 0)
    for t in range(SLAB // CB):
        pltpu.sync_copy(rows0, agg.at[pl.ds(s * SLAB + t * CB, CB)])
    plsc.subcore_barrier()

    for b in range(2):
        pltpu.async_copy(z_hbm.at[sall.at[b]], rows_l[b], sg_l[b])
        pltpu.async_copy(dst3.at[wid, b], dring.at[b], sd_l[b])
        pltpu.async_copy(wf_hbm.at[pl.ds(wbase + b * CB, CB)], wring.at[b],
                         sw_l[b])

    def group(j, _):
        for b in range(4):
            k = 4 * j + b
            rb = rows_l[b]
            pltpu.make_async_copy(z_hbm.at[sall.at[k]], rb, sg_l[b]).wait()
            pltpu.make_async_copy(wf_hbm.at[pl.ds(wbase, CB)], wring.at[b],
                                  sw_l[b]).wait()

            bvec = jnp.full((16,), b, _i32)

            def scale(e, _):
                ev = jnp.zeros((16,), _i32) + e
                w16 = plsc.load_gather(wring, [bvec, ev])
                for q in range(nq):
                    rb[e, pl.ds(q * 16, 16)] = rb[e, pl.ds(q * 16, 16)] * w16
                return 0
            lax.fori_loop(0, CB, scale, 0)

            pltpu.make_async_copy(dst3.at[wid, k], dring.at[b],
                                  sd_l[b]).wait()
            pltpu.async_copy(rb, agg.at[dring.at[b]], ss_l[b], add=True)

            b2 = (b + 2) % 4

            @pl.when(k >= 2)
            def _():
                pltpu.make_async_copy(rows_l[b2], agg.at[dring.at[b2]],
                                      ss_l[b2]).wait()

            @pl.when(k < NCHUNK - 2)
            def _():
                pltpu.async_copy(z_hbm.at[sall.at[k + 2]], rows_l[b2],
                                 sg_l[b2])
                pltpu.async_copy(dst3.at[wid, k + 2], dring.at[b2],
                                 sd_l[b2])
                pltpu.async_copy(
                    wf_hbm.at[pl.ds(wbase + (k + 2) * CB, CB)],
                    wring.at[b2], sw_l[b2])
        return 0
    lax.fori_loop(0, NCHUNK // 4, group, 0)

    pltpu.make_async_copy(rows2, agg.at[dring.at[2]], ss2).wait()
    pltpu.make_async_copy(rows3, agg.at[dring.at[3]], ss3).wait()

    plsc.subcore_barrier()
    for t in range(SLAB // CB):
        pltpu.sync_copy(agg.at[pl.ds(s * SLAB + t * CB, CB)],
                        out_hbm.at[c, pl.ds(s * SLAB + t * CB, CB)])


def _msg_call(z, src3, dst3, wf, D):
    return pl.kernel(
        functools.partial(_msg_body, D),
        out_type=jax.ShapeDtypeStruct((NCORES, NPAD, D), _f32),
        mesh=_sc_mesh,
        compiler_params=_sc_params,
        scratch_types=[
            pltpu.VMEM((NCHUNK, CB), _i32),
            pltpu.VMEM((4, CB), _i32),
            pltpu.VMEM((4, CB), _f32),
            pltpu.VMEM((CB, D), _f32),
            pltpu.VMEM((CB, D), _f32),
            pltpu.VMEM((CB, D), _f32),
            pltpu.VMEM((CB, D), _f32),
        ] + [pltpu.SemaphoreType.DMA] * 16 + [
            pltpu.VMEM_SHARED((NPAD, D), _f32),
        ],
    )(z, src3, dst3, wf)


def _norm_kernel(dref, oref):
    d = jnp.sum(dref[...], axis=0)
    oref[...] = lax.rsqrt(jnp.clip(d, 1.0, None))


def _mm_kernel(xref, wref, oref):
    oref[...] = jnp.dot(xref[...], wref[...], preferred_element_type=_f32)


def _stats_kernel(p0, p1, aref, sref):
    i = pl.program_id(0)
    a = p0[...] + p1[...]
    a = jnp.where(a >= 0.0, a, 0.01 * a)
    aref[...] = a

    @pl.when(i == 0)
    def _():
        sref[...] = jnp.zeros_like(sref)

    sref[0:1, :] = sref[0:1, :] + jnp.sum(a, axis=0, keepdims=True)
    sref[1:2, :] = sref[1:2, :] + jnp.sum(a * a, axis=0, keepdims=True)


def _gnmm_kernel(aref, sref, gref, alref, bref, wref, oref):
    inv_n = jnp.float32(1.0 / N)
    mu = sref[0:1, :] * inv_n
    e2 = sref[1:2, :] * inv_n
    al = alref[...]
    var = e2 - (2.0 * al - al * al) * mu * mu
    sc = gref[...] * lax.rsqrt(var + 1e-5)
    t = bref[...] - sc * al * mu
    h = aref[...] * sc + t
    oref[...] = jnp.dot(h, wref[...], preferred_element_type=_f32)


def _final_kernel(p0, p1, gref, alref, bref, wcref, oref, acc):
    i = pl.program_id(0)
    nb = NPAD // 256
    a = p0[...] + p1[...]
    a = jnp.where(a >= 0.0, a, 0.01 * a)
    rid = i * 256 + lax.broadcasted_iota(_i32, (256, 1), 0)
    valid = rid < N
    big = jnp.float32(3.0e38)
    amax = jnp.where(valid, a, -big)
    amin = jnp.where(valid, a, big)

    @pl.when(i == 0)
    def _():
        acc[0:2, :] = jnp.zeros((2, D2), _f32)
        acc[2:3, :] = jnp.full((1, D2), -big, _f32)
        acc[3:4, :] = jnp.full((1, D2), big, _f32)

    acc[0:1, :] = acc[0:1, :] + jnp.sum(a, axis=0, keepdims=True)
    acc[1:2, :] = acc[1:2, :] + jnp.sum(a * a, axis=0, keepdims=True)
    acc[2:3, :] = jnp.maximum(acc[2:3, :], jnp.max(amax, axis=0, keepdims=True))
    acc[3:4, :] = jnp.minimum(acc[3:4, :], jnp.min(amin, axis=0, keepdims=True))

    @pl.when(i == nb - 1)
    def _():
        inv_n = jnp.float32(1.0 / N)
        mu = acc[0:1, :] * inv_n
        e2 = acc[1:2, :] * inv_n
        al = alref[...]
        var = e2 - (2.0 * al - al * al) * mu * mu
        sc = gref[...] * lax.rsqrt(var + 1e-5)
        t = bref[...] - sc * al * mu
        pos = sc >= 0.0
        meanh = sc * mu + t
        maxh = jnp.where(pos, sc * acc[2:3, :], sc * acc[3:4, :]) + t
        minh = jnp.where(pos, sc * acc[3:4, :], sc * acc[2:3, :]) + t
        feats = jnp.concatenate([meanh, maxh, minh], axis=1)
        oref[...] = jnp.dot(feats, wcref[...], preferred_element_type=_f32)


def _row_spec(d):
    return pl.BlockSpec((256, d), lambda i: (i, 0))


def _const_spec(shape):
    return pl.BlockSpec(shape, lambda i: tuple(0 for _ in shape))


def kernel(x, edge_index, edge_weights, W1, W2, Wc,
           gn1_alpha, gn1_gamma, gn1_beta,
           gn2_alpha, gn2_gamma, gn2_beta):
    src = edge_index[0].astype(_i32)
    dst = edge_index[1].astype(_i32)
    padi = jnp.full((EPAD - E,), N, _i32)
    srcp = jnp.concatenate([src, padi])
    dstp = jnp.concatenate([dst, padi])
    ewp = jnp.concatenate([edge_weights.astype(_f32),
                           jnp.zeros((EPAD - E,), _f32)])
    xp = jnp.pad(x, ((0, NPAD - N), (0, 0)))

    degs = _deg_call(srcp, dstp)
    norms = pl.pallas_call(
        _norm_kernel,
        out_shape=jax.ShapeDtypeStruct((2, NPAD // 128, 128), _f32),
    )(degs.reshape(NW, 2, NPAD // 128, 128))
    no = norms[0].reshape(NPAD)
    ni = norms[1].reshape(NPAD)

    wf = _w_call(srcp, dstp, ewp, no, ni)
    src3 = srcp.reshape(NW, NCHUNK, CB)
    dst3 = dstp.reshape(NW, NCHUNK, CB)

    nb = NPAD // 256
    z1 = pl.pallas_call(
        _mm_kernel,
        grid=(nb,),
        in_specs=[_row_spec(D1), _const_spec((D1, D1))],
        out_specs=_row_spec(D1),
        out_shape=jax.ShapeDtypeStruct((NPAD, D1), _f32),
    )(xp, W1)

    p1 = _msg_call(z1, src3, dst3, wf, D1)

    a1, st1 = pl.pallas_call(
        _stats_kernel,
        grid=(nb,),
        in_specs=[_row_spec(D1), _row_spec(D1)],
        out_specs=[_row_spec(D1), _const_spec((8, D1))],
        out_shape=[jax.ShapeDtypeStruct((NPAD, D1), _f32),
                   jax.ShapeDtypeStruct((8, D1), _f32)],
    )(p1[0], p1[1])

    z2 = pl.pallas_call(
        _gnmm_kernel,
        grid=(nb,),
        in_specs=[_row_spec(D1), _const_spec((8, D1)), _const_spec((1, D1)),
                  _const_spec((1, D1)), _const_spec((1, D1)),
                  _const_spec((D1, D2))],
        out_specs=_row_spec(D2),
        out_shape=jax.ShapeDtypeStruct((NPAD, D2), _f32),
    )(a1, st1, gn1_gamma.reshape(1, D1), gn1_alpha.reshape(1, D1),
      gn1_beta.reshape(1, D1), W2)

    p2 = _msg_call(z2, src3, dst3, wf, D2)

    out = pl.pallas_call(
        _final_kernel,
        grid=(nb,),
        in_specs=[_row_spec(D2), _row_spec(D2), _const_spec((1, D2)),
                  _const_spec((1, D2)), _const_spec((1, D2)),
                  _const_spec((3 * D2, OUT_DIM))],
        out_specs=_const_spec((1, OUT_DIM)),
        out_shape=jax.ShapeDtypeStruct((1, OUT_DIM), _f32),
        scratch_shapes=[pltpu.VMEM((8, D2), _f32)],
    )(p2[0], p2[1], gn2_gamma.reshape(1, D2), gn2_alpha.reshape(1, D2),
      gn2_beta.reshape(1, D2), Wc)

    return out


# speedup vs baseline: 6.9084x; 1.6009x over previous
import functools

import jax
import jax.numpy as jnp
from jax import lax
from jax.experimental import pallas as pl
from jax.experimental.pallas import tpu as pltpu
from jax.experimental.pallas import tpu_sc as plsc

N = 10000
NPAD = 10240
E = 320000
D1 = 128
D2 = 64
OUT_DIM = 10
NCORES = 2
NSUB = 16
NW = NCORES * NSUB
EPT = 10240
EPAD = NW * EPT
CB = 64
NCHUNK = EPT // CB
ROWS = NPAD // 16
SLAB = NPAD // NSUB

_f32 = jnp.float32
_i32 = jnp.int32

_sc_mesh = plsc.VectorSubcoreMesh(core_axis_name="c", subcore_axis_name="s")
_sc_params = pltpu.CompilerParams(needs_layout_passes=False,
                                  use_tc_tiling_on_sc=False)


def _deg_body(src_hbm, dst_hbm, out_hbm, hs, hd, idx_s, idx_d):
    c = lax.axis_index("c")
    s = lax.axis_index("s")
    wid = c * NSUB + s

    def zloop(j, _):
        hs[pl.ds(j * 16, 16)] = jnp.zeros((16,), _f32)
        hd[pl.ds(j * 16, 16)] = jnp.zeros((16,), _f32)
        return 0
    lax.fori_loop(0, NPAD // 16, zloop, 0)

    ones = jnp.full((16,), 1.0, _f32)
    CA = 1024
    base = wid * EPT

    def chunk(k, _):
        pltpu.sync_copy(src_hbm.at[pl.ds(base + k * CA, CA)], idx_s)
        pltpu.sync_copy(dst_hbm.at[pl.ds(base + k * CA, CA)], idx_d)

        def inner(j, _):
            sv = idx_s[pl.ds(j * 16, 16)]
            dv = idx_d[pl.ds(j * 16, 16)]
            plsc.addupdate_scatter(hs, [sv], ones)
            plsc.addupdate_scatter(hd, [dv], ones)
            return 0
        lax.fori_loop(0, CA // 16, inner, 0)
        return 0
    lax.fori_loop(0, EPT // CA, chunk, 0)

    pltpu.sync_copy(hs, out_hbm.at[wid, 0])
    pltpu.sync_copy(hd, out_hbm.at[wid, 1])


def _deg_call(srcp, dstp):
    return pl.kernel(
        _deg_body,
        out_type=jax.ShapeDtypeStruct((NW, 2, NPAD), _f32),
        mesh=_sc_mesh,
        compiler_params=_sc_params,
        scratch_types=[
            pltpu.VMEM((NPAD,), _f32),
            pltpu.VMEM((NPAD,), _f32),
            pltpu.VMEM((1024,), _i32),
            pltpu.VMEM((1024,), _i32),
        ],
    )(srcp, dstp)


def _w_body(src_hbm, dst_hbm, ew_hbm, no_hbm, ni_hbm, out_hbm,
            no_v, ni_v, s_all, d_all, e_all, w_all):
    c = lax.axis_index("c")
    s = lax.axis_index("s")
    wid = c * NSUB + s
    base = wid * EPT

    pltpu.sync_copy(no_hbm, no_v)
    pltpu.sync_copy(ni_hbm, ni_v)
    pltpu.sync_copy(src_hbm.at[pl.ds(base, EPT)], s_all)
    pltpu.sync_copy(dst_hbm.at[pl.ds(base, EPT)], d_all)
    pltpu.sync_copy(ew_hbm.at[pl.ds(base, EPT)], e_all)

    def wcomp(j, _):
        sv = s_all[pl.ds(j * 16, 16)]
        dv = d_all[pl.ds(j * 16, 16)]
        nov = plsc.load_gather(no_v, [sv])
        niv = plsc.load_gather(ni_v, [dv])
        w_all[pl.ds(j * 16, 16)] = e_all[pl.ds(j * 16, 16)] * nov * niv
        return 0
    lax.fori_loop(0, EPT // 16, wcomp, 0)

    pltpu.sync_copy(w_all, out_hbm.at[pl.ds(base, EPT)])


def _w_call(srcp, dstp, ewp, no, ni):
    return pl.kernel(
        _w_body,
        out_type=jax.ShapeDtypeStruct((EPAD,), _f32),
        mesh=_sc_mesh,
        compiler_params=_sc_params,
        scratch_types=[
            pltpu.VMEM((NPAD,), _f32),
            pltpu.VMEM((NPAD,), _f32),
            pltpu.VMEM((EPT,), _i32),
            pltpu.VMEM((EPT,), _i32),
            pltpu.VMEM((EPT,), _f32),
            pltpu.VMEM((EPT,), _f32),
        ],
    )(srcp, dstp, ewp, no, ni)


MCB = 128
NCH_TOT = EPAD // MCB

def _msg_body(feature_split, z_a, z_b, comb_hbm, out_hbm,
              ring, rows0, rows1, rows2, rows3,
              sg0, sg1, sg2, sg3, ss0, ss1, ss2, ss3,
              sr0, sr1, sr2, sr3, zsp, agg):
    c = lax.axis_index("c")
    s = lax.axis_index("s")
    rows_l = (rows0, rows1, rows2, rows3)
    sg_l = (sg0, sg1, sg2, sg3)
    ss_l = (ss0, ss1, ss2, ss3)
    sr_l = (sr0, sr1, sr2, sr3)
    if feature_split:
        nch = NCH_TOT // NSUB
        start = s * nch
    else:
        nch = NCH_TOT // NW
        start = (c * NSUB + s) * nch

    @pl.when(c == 0)
    def _():
        pltpu.sync_copy(z_a.at[pl.ds(s * SLAB, SLAB)],
                        zsp.at[pl.ds(s * SLAB, SLAB)])

    @pl.when(c == 1)
    def _():
        pltpu.sync_copy(z_b.at[pl.ds(s * SLAB, SLAB)],
                        zsp.at[pl.ds(s * SLAB, SLAB)])

    def zloop(j, _):
        for q in range(4):
            rows0[j, pl.ds(q * 16, 16)] = jnp.zeros((16,), _f32)
        return 0
    lax.fori_loop(0, MCB, zloop, 0)
    for t in range(SLAB // MCB):
        pltpu.sync_copy(rows0, agg.at[pl.ds(s * SLAB + t * MCB, MCB)])
    plsc.subcore_barrier()

    for b in range(2):
        pltpu.async_copy(comb_hbm.at[start + b], ring.at[b], sr_l[b])
    for b in range(2):
        pltpu.make_async_copy(comb_hbm.at[start + b], ring.at[b],
                              sr_l[b]).wait()
        pltpu.async_copy(zsp.at[ring.at[b, 0]], rows_l[b], sg_l[b])

    def group(j, _):
        for b in range(4):
            k = 4 * j + b
            rb = rows_l[b]
            pltpu.make_async_copy(zsp.at[ring.at[b, 0]], rb, sg_l[b]).wait()

            bvec = jnp.full((16,), b, _i32)
            two = jnp.full((16,), 2, _i32)

            def scale(h, _):
                for u in range(2):
                    e = h * 2 + u
                    ev = jnp.zeros((16,), _i32) + e
                    w16 = plsc.bitcast(
                        plsc.load_gather(ring, [bvec, two, ev]), _f32)
                    for q in range(4):
                        rb[e, pl.ds(q * 16, 16)] = (
                            rb[e, pl.ds(q * 16, 16)] * w16)
                return 0
            lax.fori_loop(0, MCB // 2, scale, 0)

            pltpu.async_copy(rb, agg.at[ring.at[b, 1]], ss_l[b], add=True)

            b2 = (b + 2) % 4

            @pl.when(k >= 2)
            def _():
                pltpu.make_async_copy(rows_l[b2], agg.at[ring.at[b2, 1]],
                                      ss_l[b2]).wait()

            @pl.when(k < nch - 2)
            def _():
                pltpu.async_copy(comb_hbm.at[start + k + 2], ring.at[b2],
                                 sr_l[b2])
                pltpu.make_async_copy(comb_hbm.at[start + k + 2],
                                      ring.at[b2], sr_l[b2]).wait()
                pltpu.async_copy(zsp.at[ring.at[b2, 0]], rows_l[b2],
                                 sg_l[b2])
        return 0
    lax.fori_loop(0, nch // 4, group, 0)

    pltpu.make_async_copy(rows2, agg.at[ring.at[2, 1]], ss2).wait()
    pltpu.make_async_copy(rows3, agg.at[ring.at[3, 1]], ss3).wait()

    plsc.subcore_barrier()
    for t in range(SLAB // MCB):
        pltpu.sync_copy(agg.at[pl.ds(s * SLAB + t * MCB, MCB)],
                        out_hbm.at[c, pl.ds(s * SLAB + t * MCB, MCB)])


def _msg_call(z_a, z_b, comb, feature_split):
    return pl.kernel(
        functools.partial(_msg_body, feature_split),
        out_type=jax.ShapeDtypeStruct((NCORES, NPAD, D2), _f32),
        mesh=_sc_mesh,
        compiler_params=_sc_params,
        scratch_types=[
            pltpu.VMEM((4, 3, MCB), _i32),
            pltpu.VMEM((MCB, D2), _f32),
            pltpu.VMEM((MCB, D2), _f32),
            pltpu.VMEM((MCB, D2), _f32),
            pltpu.VMEM((MCB, D2), _f32),
        ] + [pltpu.SemaphoreType.DMA] * 12 + [
            pltpu.VMEM_SHARED((NPAD, D2), _f32),
            pltpu.VMEM_SHARED((NPAD, D2), _f32),
        ],
    )(z_a, z_b, comb)


def _norm_kernel(dref, oref):
    d = jnp.sum(dref[...], axis=0)
    oref[...] = lax.rsqrt(jnp.clip(d, 1.0, None))


def _mm_kernel(xref, wref, oref):
    oref[...] = jnp.dot(xref[...], wref[...], preferred_element_type=_f32)


def _stats_kernel(p0, aref, sref):
    i = pl.program_id(0)
    a = p0[...]
    a = jnp.where(a >= 0.0, a, 0.01 * a)
    aref[...] = a

    @pl.when(i == 0)
    def _():
        sref[...] = jnp.zeros_like(sref)

    sref[0:1, :] = sref[0:1, :] + jnp.sum(a, axis=0, keepdims=True)
    sref[1:2, :] = sref[1:2, :] + jnp.sum(a * a, axis=0, keepdims=True)


def _gnmm_kernel(aref, sref, gref, alref, bref, wref, oref):
    inv_n = jnp.float32(1.0 / N)
    mu = sref[0:1, :] * inv_n
    e2 = sref[1:2, :] * inv_n
    al = alref[...]
    var = e2 - (2.0 * al - al * al) * mu * mu
    sc = gref[...] * lax.rsqrt(var + 1e-5)
    t = bref[...] - sc * al * mu
    h = aref[...] * sc + t
    oref[...] = jnp.dot(h, wref[...], preferred_element_type=_f32)


def _final_kernel(p0, p1, gref, alref, bref, wcref, oref, acc):
    i = pl.program_id(0)
    nb = NPAD // 256
    a = p0[...] + p1[...]
    a = jnp.where(a >= 0.0, a, 0.01 * a)
    rid = i * 256 + lax.broadcasted_iota(_i32, (256, 1), 0)
    valid = rid < N
    big = jnp.float32(3.0e38)
    amax = jnp.where(valid, a, -big)
    amin = jnp.where(valid, a, big)

    @pl.when(i == 0)
    def _():
        acc[0:2, :] = jnp.zeros((2, D2), _f32)
        acc[2:3, :] = jnp.full((1, D2), -big, _f32)
        acc[3:4, :] = jnp.full((1, D2), big, _f32)

    acc[0:1, :] = acc[0:1, :] + jnp.sum(a, axis=0, keepdims=True)
    acc[1:2, :] = acc[1:2, :] + jnp.sum(a * a, axis=0, keepdims=True)
    acc[2:3, :] = jnp.maximum(acc[2:3, :], jnp.max(amax, axis=0, keepdims=True))
    acc[3:4, :] = jnp.minimum(acc[3:4, :], jnp.min(amin, axis=0, keepdims=True))

    @pl.when(i == nb - 1)
    def _():
        inv_n = jnp.float32(1.0 / N)
        mu = acc[0:1, :] * inv_n
        e2 = acc[1:2, :] * inv_n
        al = alref[...]
        var = e2 - (2.0 * al - al * al) * mu * mu
        sc = gref[...] * lax.rsqrt(var + 1e-5)
        t = bref[...] - sc * al * mu
        pos = sc >= 0.0
        meanh = sc * mu + t
        maxh = jnp.where(pos, sc * acc[2:3, :], sc * acc[3:4, :]) + t
        minh = jnp.where(pos, sc * acc[3:4, :], sc * acc[2:3, :]) + t
        feats = jnp.concatenate([meanh, maxh, minh], axis=1)
        oref[...] = jnp.dot(feats, wcref[...], preferred_element_type=_f32)


def _row_spec(d):
    return pl.BlockSpec((256, d), lambda i: (i, 0))


def _const_spec(shape):
    return pl.BlockSpec(shape, lambda i: tuple(0 for _ in shape))


def kernel(x, edge_index, edge_weights, W1, W2, Wc,
           gn1_alpha, gn1_gamma, gn1_beta,
           gn2_alpha, gn2_gamma, gn2_beta):
    src = edge_index[0].astype(_i32)
    dst = edge_index[1].astype(_i32)
    padi = jnp.full((EPAD - E,), N, _i32)
    srcp = jnp.concatenate([src, padi])
    dstp = jnp.concatenate([dst, padi])
    ewp = jnp.concatenate([edge_weights.astype(_f32),
                           jnp.zeros((EPAD - E,), _f32)])
    xp = jnp.pad(x, ((0, NPAD - N), (0, 0)))

    degs = _deg_call(srcp, dstp)
    norms = pl.pallas_call(
        _norm_kernel,
        out_shape=jax.ShapeDtypeStruct((2, NPAD // 128, 128), _f32),
    )(degs.reshape(NW, 2, NPAD // 128, 128))
    no = norms[0].reshape(NPAD)
    ni = norms[1].reshape(NPAD)

    wf = _w_call(srcp, dstp, ewp, no, ni)
    comb = jnp.stack([srcp.reshape(NCH_TOT, MCB),
                      dstp.reshape(NCH_TOT, MCB),
                      lax.bitcast_convert_type(wf, _i32).reshape(NCH_TOT, MCB)],
                     axis=1)

    nb = NPAD // 256
    z1 = pl.pallas_call(
        _mm_kernel,
        grid=(nb,),
        in_specs=[_row_spec(D1), _const_spec((D1, D1))],
        out_specs=_row_spec(D1),
        out_shape=jax.ShapeDtypeStruct((NPAD, D1), _f32),
    )(xp, W1)

    p1 = _msg_call(z1[:, :D2], z1[:, D2:], comb, True)
    m1 = jnp.concatenate([p1[0], p1[1]], axis=1)

    a1, st1 = pl.pallas_call(
        _stats_kernel,
        grid=(nb,),
        in_specs=[_row_spec(D1)],
        out_specs=[_row_spec(D1), _const_spec((8, D1))],
        out_shape=[jax.ShapeDtypeStruct((NPAD, D1), _f32),
                   jax.ShapeDtypeStruct((8, D1), _f32)],
    )(m1)

    z2 = pl.pallas_call(
        _gnmm_kernel,
        grid=(nb,),
        in_specs=[_row_spec(D1), _const_spec((8, D1)), _const_spec((1, D1)),
                  _const_spec((1, D1)), _const_spec((1, D1)),
                  _const_spec((D1, D2))],
        out_specs=_row_spec(D2),
        out_shape=jax.ShapeDtypeStruct((NPAD, D2), _f32),
    )(a1, st1, gn1_gamma.reshape(1, D1), gn1_alpha.reshape(1, D1),
      gn1_beta.reshape(1, D1), W2)

    p2 = _msg_call(z2, z2, comb, False)

    out = pl.pallas_call(
        _final_kernel,
        grid=(nb,),
        in_specs=[_row_spec(D2), _row_spec(D2), _const_spec((1, D2)),
                  _const_spec((1, D2)), _const_spec((1, D2)),
                  _const_spec((3 * D2, OUT_DIM))],
        out_specs=_const_spec((1, OUT_DIM)),
        out_shape=jax.ShapeDtypeStruct((1, OUT_DIM), _f32),
        scratch_shapes=[pltpu.VMEM((8, D2), _f32)],
    )(p2[0], p2[1], gn2_gamma.reshape(1, D2), gn2_alpha.reshape(1, D2),
      gn2_beta.reshape(1, D2), Wc)

    return out


# speedup vs baseline: 7.6679x; 1.1099x over previous
import functools

import jax
import jax.numpy as jnp
from jax import lax
from jax.experimental import pallas as pl
from jax.experimental.pallas import tpu as pltpu
from jax.experimental.pallas import tpu_sc as plsc

N = 10000
NPAD = 10240
E = 320000
D1 = 128
D2 = 64
OUT_DIM = 10
NCORES = 2
NSUB = 16
NW = NCORES * NSUB
EPT = 10240
EPAD = NW * EPT
CB = 64
NCHUNK = EPT // CB
ROWS = NPAD // 16
SLAB = NPAD // NSUB

_f32 = jnp.float32
_i32 = jnp.int32

_sc_mesh = plsc.VectorSubcoreMesh(core_axis_name="c", subcore_axis_name="s")
_sc_params = pltpu.CompilerParams(needs_layout_passes=False,
                                  use_tc_tiling_on_sc=False)


def _deg_body(src_hbm, dst_hbm, out_hbm, hs, hd, idx_s, idx_d):
    c = lax.axis_index("c")
    s = lax.axis_index("s")
    wid = c * NSUB + s

    def zloop(j, _):
        hs[pl.ds(j * 16, 16)] = jnp.zeros((16,), _f32)
        hd[pl.ds(j * 16, 16)] = jnp.zeros((16,), _f32)
        return 0
    lax.fori_loop(0, NPAD // 16, zloop, 0)

    ones = jnp.full((16,), 1.0, _f32)
    CA = 1024
    base = wid * EPT

    def chunk(k, _):
        pltpu.sync_copy(src_hbm.at[pl.ds(base + k * CA, CA)], idx_s)
        pltpu.sync_copy(dst_hbm.at[pl.ds(base + k * CA, CA)], idx_d)

        def inner(j, _):
            sv = idx_s[pl.ds(j * 16, 16)]
            dv = idx_d[pl.ds(j * 16, 16)]
            plsc.addupdate_scatter(hs, [sv], ones)
            plsc.addupdate_scatter(hd, [dv], ones)
            return 0
        lax.fori_loop(0, CA // 16, inner, 0)
        return 0
    lax.fori_loop(0, EPT // CA, chunk, 0)

    pltpu.sync_copy(hs, out_hbm.at[wid, 0])
    pltpu.sync_copy(hd, out_hbm.at[wid, 1])


def _deg_call(srcp, dstp):
    return pl.kernel(
        _deg_body,
        out_type=jax.ShapeDtypeStruct((NW, 2, NPAD), _f32),
        mesh=_sc_mesh,
        compiler_params=_sc_params,
        scratch_types=[
            pltpu.VMEM((NPAD,), _f32),
            pltpu.VMEM((NPAD,), _f32),
            pltpu.VMEM((1024,), _i32),
            pltpu.VMEM((1024,), _i32),
        ],
    )(srcp, dstp)


def _w_body(src_hbm, dst_hbm, ew_hbm, no_hbm, ni_hbm, out_hbm,
            no_v, ni_v, s_all, d_all, e_all, w_all):
    c = lax.axis_index("c")
    s = lax.axis_index("s")
    wid = c * NSUB + s
    base = wid * EPT

    pltpu.sync_copy(no_hbm, no_v)
    pltpu.sync_copy(ni_hbm, ni_v)
    pltpu.sync_copy(src_hbm.at[pl.ds(base, EPT)], s_all)
    pltpu.sync_copy(dst_hbm.at[pl.ds(base, EPT)], d_all)
    pltpu.sync_copy(ew_hbm.at[pl.ds(base, EPT)], e_all)

    def wcomp(j, _):
        sv = s_all[pl.ds(j * 16, 16)]
        dv = d_all[pl.ds(j * 16, 16)]
        nov = plsc.load_gather(no_v, [sv])
        niv = plsc.load_gather(ni_v, [dv])
        w_all[pl.ds(j * 16, 16)] = e_all[pl.ds(j * 16, 16)] * nov * niv
        return 0
    lax.fori_loop(0, EPT // 16, wcomp, 0)

    pltpu.sync_copy(w_all, out_hbm.at[pl.ds(base, EPT)])


def _w_call(srcp, dstp, ewp, no, ni):
    return pl.kernel(
        _w_body,
        out_type=jax.ShapeDtypeStruct((EPAD,), _f32),
        mesh=_sc_mesh,
        compiler_params=_sc_params,
        scratch_types=[
            pltpu.VMEM((NPAD,), _f32),
            pltpu.VMEM((NPAD,), _f32),
            pltpu.VMEM((EPT,), _i32),
            pltpu.VMEM((EPT,), _i32),
            pltpu.VMEM((EPT,), _f32),
            pltpu.VMEM((EPT,), _f32),
        ],
    )(srcp, dstp, ewp, no, ni)


MCB = 128
NCH_TOT = EPAD // MCB

def _msg_body(feature_split, z_a, z_b, comb_hbm, out_hbm,
              ring, rows0, rows1, rows2, rows3,
              sg0, sg1, sg2, sg3, ss0, ss1, ss2, ss3,
              sr0, sr1, sr2, sr3, zsp, agg):
    c = lax.axis_index("c")
    s = lax.axis_index("s")
    rows_l = (rows0, rows1, rows2, rows3)
    sg_l = (sg0, sg1, sg2, sg3)
    ss_l = (ss0, ss1, ss2, ss3)
    sr_l = (sr0, sr1, sr2, sr3)
    if feature_split:
        nch = NCH_TOT // NSUB
        start = s * nch
    else:
        nch = NCH_TOT // NW
        start = (c * NSUB + s) * nch

    @pl.when(c == 0)
    def _():
        pltpu.sync_copy(z_a.at[pl.ds(s * SLAB, SLAB)],
                        zsp.at[pl.ds(s * SLAB, SLAB)])

    @pl.when(c == 1)
    def _():
        pltpu.sync_copy(z_b.at[pl.ds(s * SLAB, SLAB)],
                        zsp.at[pl.ds(s * SLAB, SLAB)])

    def zloop(j, _):
        for q in range(4):
            rows0[j, pl.ds(q * 16, 16)] = jnp.zeros((16,), _f32)
        return 0
    lax.fori_loop(0, MCB, zloop, 0)
    for t in range(SLAB // MCB):
        pltpu.sync_copy(rows0, agg.at[pl.ds(s * SLAB + t * MCB, MCB)])
    plsc.subcore_barrier()

    for b in range(2):
        pltpu.async_copy(comb_hbm.at[start + b], ring.at[b], sr_l[b])
    for b in range(2):
        pltpu.make_async_copy(comb_hbm.at[start + b], ring.at[b],
                              sr_l[b]).wait()
        pltpu.async_copy(zsp.at[ring.at[b, 0]], rows_l[b], sg_l[b])

    def group(j, _):
        for b in range(4):
            k = 4 * j + b
            rb = rows_l[b]
            pltpu.make_async_copy(zsp.at[ring.at[b, 0]], rb, sg_l[b]).wait()

            bvec = jnp.full((16,), b, _i32)
            two = jnp.full((16,), 2, _i32)

            def scale(h, _):
                for u in range(4):
                    e = h * 4 + u
                    ev = jnp.zeros((16,), _i32) + e
                    w16 = plsc.bitcast(
                        plsc.load_gather(ring, [bvec, two, ev]), _f32)
                    for q in range(4):
                        rb[e, pl.ds(q * 16, 16)] = (
                            rb[e, pl.ds(q * 16, 16)] * w16)
                return 0
            lax.fori_loop(0, MCB // 4, scale, 0)

            pltpu.async_copy(rb, agg.at[ring.at[b, 1]], ss_l[b], add=True)

            b2 = (b + 2) % 4

            @pl.when(k >= 2)
            def _():
                pltpu.make_async_copy(rows_l[b2], agg.at[ring.at[b2, 1]],
                                      ss_l[b2]).wait()

            @pl.when(k < nch - 2)
            def _():
                pltpu.async_copy(comb_hbm.at[start + k + 2], ring.at[b2],
                                 sr_l[b2])
                pltpu.make_async_copy(comb_hbm.at[start + k + 2],
                                      ring.at[b2], sr_l[b2]).wait()
                pltpu.async_copy(zsp.at[ring.at[b2, 0]], rows_l[b2],
                                 sg_l[b2])
        return 0
    lax.fori_loop(0, nch // 4, group, 0)

    pltpu.make_async_copy(rows2, agg.at[ring.at[2, 1]], ss2).wait()
    pltpu.make_async_copy(rows3, agg.at[ring.at[3, 1]], ss3).wait()

    plsc.subcore_barrier()
    for t in range(SLAB // MCB):
        pltpu.sync_copy(agg.at[pl.ds(s * SLAB + t * MCB, MCB)],
                        out_hbm.at[c, pl.ds(s * SLAB + t * MCB, MCB)])


def _msg_call(z_a, z_b, comb, feature_split):
    return pl.kernel(
        functools.partial(_msg_body, feature_split),
        out_type=jax.ShapeDtypeStruct((NCORES, NPAD, D2), _f32),
        mesh=_sc_mesh,
        compiler_params=_sc_params,
        scratch_types=[
            pltpu.VMEM((4, 3, MCB), _i32),
            pltpu.VMEM((MCB, D2), _f32),
            pltpu.VMEM((MCB, D2), _f32),
            pltpu.VMEM((MCB, D2), _f32),
            pltpu.VMEM((MCB, D2), _f32),
        ] + [pltpu.SemaphoreType.DMA] * 12 + [
            pltpu.VMEM_SHARED((NPAD, D2), _f32),
            pltpu.VMEM_SHARED((NPAD, D2), _f32),
        ],
    )(z_a, z_b, comb)


def _norm_kernel(dref, oref):
    d = jnp.sum(dref[...], axis=0)
    oref[...] = lax.rsqrt(jnp.clip(d, 1.0, None))


def _mm_kernel(xref, wref, oref):
    oref[...] = jnp.dot(xref[...], wref[...], preferred_element_type=_f32)


def _stats_kernel(pa, pb, aref, sref):
    i = pl.program_id(0)
    a = jnp.concatenate([pa[...], pb[...]], axis=1)
    a = jnp.where(a >= 0.0, a, 0.01 * a)
    aref[...] = a

    @pl.when(i == 0)
    def _():
        sref[...] = jnp.zeros_like(sref)

    sref[0:1, :] = sref[0:1, :] + jnp.sum(a, axis=0, keepdims=True)
    sref[1:2, :] = sref[1:2, :] + jnp.sum(a * a, axis=0, keepdims=True)


def _gnmm_kernel(aref, sref, gref, alref, bref, wref, oref):
    inv_n = jnp.float32(1.0 / N)
    mu = sref[0:1, :] * inv_n
    e2 = sref[1:2, :] * inv_n
    al = alref[...]
    var = e2 - (2.0 * al - al * al) * mu * mu
    sc = gref[...] * lax.rsqrt(var + 1e-5)
    t = bref[...] - sc * al * mu
    h = aref[...] * sc + t
    oref[...] = jnp.dot(h, wref[...], preferred_element_type=_f32)


def _final_kernel(p0, p1, gref, alref, bref, wcref, oref, acc):
    i = pl.program_id(0)
    nb = NPAD // RB
    a = p0[...] + p1[...]
    a = jnp.where(a >= 0.0, a, 0.01 * a)
    rid = i * RB + lax.broadcasted_iota(_i32, (RB, 1), 0)
    valid = rid < N
    big = jnp.float32(3.0e38)
    amax = jnp.where(valid, a, -big)
    amin = jnp.where(valid, a, big)

    @pl.when(i == 0)
    def _():
        acc[0:2, :] = jnp.zeros((2, D2), _f32)
        acc[2:3, :] = jnp.full((1, D2), -big, _f32)
        acc[3:4, :] = jnp.full((1, D2), big, _f32)

    acc[0:1, :] = acc[0:1, :] + jnp.sum(a, axis=0, keepdims=True)
    acc[1:2, :] = acc[1:2, :] + jnp.sum(a * a, axis=0, keepdims=True)
    acc[2:3, :] = jnp.maximum(acc[2:3, :], jnp.max(amax, axis=0, keepdims=True))
    acc[3:4, :] = jnp.minimum(acc[3:4, :], jnp.min(amin, axis=0, keepdims=True))

    @pl.when(i == nb - 1)
    def _():
        inv_n = jnp.float32(1.0 / N)
        mu = acc[0:1, :] * inv_n
        e2 = acc[1:2, :] * inv_n
        al = alref[...]
        var = e2 - (2.0 * al - al * al) * mu * mu
        sc = gref[...] * lax.rsqrt(var + 1e-5)
        t = bref[...] - sc * al * mu
        pos = sc >= 0.0
        meanh = sc * mu + t
        maxh = jnp.where(pos, sc * acc[2:3, :], sc * acc[3:4, :]) + t
        minh = jnp.where(pos, sc * acc[3:4, :], sc * acc[2:3, :]) + t
        feats = jnp.concatenate([meanh, maxh, minh], axis=1)
        oref[...] = jnp.dot(feats, wcref[...], preferred_element_type=_f32)


RB = 1024


def _row_spec(d):
    return pl.BlockSpec((RB, d), lambda i: (i, 0))


def _const_spec(shape):
    return pl.BlockSpec(shape, lambda i: tuple(0 for _ in shape))


def kernel(x, edge_index, edge_weights, W1, W2, Wc,
           gn1_alpha, gn1_gamma, gn1_beta,
           gn2_alpha, gn2_gamma, gn2_beta):
    src = edge_index[0].astype(_i32)
    dst = edge_index[1].astype(_i32)
    padi = jnp.full((EPAD - E,), N, _i32)
    srcp = jnp.concatenate([src, padi])
    dstp = jnp.concatenate([dst, padi])
    ewp = jnp.concatenate([edge_weights.astype(_f32),
                           jnp.zeros((EPAD - E,), _f32)])
    xp = jnp.pad(x, ((0, NPAD - N), (0, 0)))

    degs = _deg_call(srcp, dstp)
    norms = pl.pallas_call(
        _norm_kernel,
        out_shape=jax.ShapeDtypeStruct((2, NPAD // 128, 128), _f32),
    )(degs.reshape(NW, 2, NPAD // 128, 128))
    no = norms[0].reshape(NPAD)
    ni = norms[1].reshape(NPAD)

    wf = _w_call(srcp, dstp, ewp, no, ni)
    comb = jnp.stack([srcp.reshape(NCH_TOT, MCB),
                      dstp.reshape(NCH_TOT, MCB),
                      lax.bitcast_convert_type(wf, _i32).reshape(NCH_TOT, MCB)],
                     axis=1)

    nb = NPAD // RB
    z1 = pl.pallas_call(
        _mm_kernel,
        grid=(nb,),
        in_specs=[_row_spec(D1), _const_spec((D1, D1))],
        out_specs=_row_spec(D1),
        out_shape=jax.ShapeDtypeStruct((NPAD, D1), _f32),
    )(xp, W1)

    p1 = _msg_call(z1[:, :D2], z1[:, D2:], comb, True)

    a1, st1 = pl.pallas_call(
        _stats_kernel,
        grid=(nb,),
        in_specs=[_row_spec(D2), _row_spec(D2)],
        out_specs=[_row_spec(D1), _const_spec((8, D1))],
        out_shape=[jax.ShapeDtypeStruct((NPAD, D1), _f32),
                   jax.ShapeDtypeStruct((8, D1), _f32)],
    )(p1[0], p1[1])

    z2 = pl.pallas_call(
        _gnmm_kernel,
        grid=(nb,),
        in_specs=[_row_spec(D1), _const_spec((8, D1)), _const_spec((1, D1)),
                  _const_spec((1, D1)), _const_spec((1, D1)),
                  _const_spec((D1, D2))],
        out_specs=_row_spec(D2),
        out_shape=jax.ShapeDtypeStruct((NPAD, D2), _f32),
    )(a1, st1, gn1_gamma.reshape(1, D1), gn1_alpha.reshape(1, D1),
      gn1_beta.reshape(1, D1), W2)

    p2 = _msg_call(z2, z2, comb, False)

    out = pl.pallas_call(
        _final_kernel,
        grid=(nb,),
        in_specs=[_row_spec(D2), _row_spec(D2), _const_spec((1, D2)),
                  _const_spec((1, D2)), _const_spec((1, D2)),
                  _const_spec((3 * D2, OUT_DIM))],
        out_specs=_const_spec((1, OUT_DIM)),
        out_shape=jax.ShapeDtypeStruct((1, OUT_DIM), _f32),
        scratch_shapes=[pltpu.VMEM((8, D2), _f32)],
    )(p2[0], p2[1], gn2_gamma.reshape(1, D2), gn2_alpha.reshape(1, D2),
      gn2_beta.reshape(1, D2), Wc)

    return out
